# baseline (speedup 1.0000x reference)
import jax
import jax.numpy as jnp
from jax import lax
from jax.experimental import pallas as pl
from jax.experimental.pallas import tpu as pltpu
from jax.experimental.pallas import tpu_sc as plsc

_N = 16384
_D = 256
_B = 8
_HID = _D // 4

_NW = 32
_RPW = _N // _NW
_CH = 128
_NCH = _RPW // _CH
_NL = 16
_DC = _D // _NL


def _sc_body(feats_hbm, seg_hbm, sums_hbm, maxs_hbm, cnts_hbm,
             seg_v, seg_sm, buf0, buf1, asum_v, amax_v, acnt_v,
             sem0, sem1):
    cc = lax.axis_index("c")
    ss = lax.axis_index("s")
    wid = ss * 2 + cc
    base = wid * _RPW

    pltpu.sync_copy(seg_hbm.at[pl.ds(base, _RPW)], seg_v)

    def _seg_to_smem(g, carry):
        vv = seg_v[pl.ds(g * _NL, _NL)]
        for k in range(_NL):
            seg_sm[g * _NL + k] = vv[k]
        return carry
    lax.fori_loop(0, _RPW // _NL, _seg_to_smem, 0)

    def _init(i, carry):
        asum_v[pl.ds(i * _NL, _NL)] = jnp.zeros((_NL,), jnp.float32)
        amax_v[pl.ds(i * _NL, _NL)] = jnp.full((_NL,), -jnp.inf,
                                               jnp.float32)
        return carry
    lax.fori_loop(0, _B * _DC, _init, 0)
    for b in range(_B):
        acnt_v[pl.ds(b * _NL, _NL)] = jnp.zeros((_NL,), jnp.float32)

    bufs = (buf0, buf1)
    sems = (sem0, sem1)
    cps = [None, None]
    cps[0] = pltpu.async_copy(
        feats_hbm.at[pl.ds(base * _D, _CH * _D)], buf0, sem0)

    carry = (seg_sm[0], jnp.asarray(0.0, jnp.float32),
             *([jnp.zeros((_NL,), jnp.float32)] * _DC),
             *([jnp.full((_NL,), -jnp.inf, jnp.float32)] * _DC))

    for cidx in range(_NCH):
        nb = (cidx + 1) % 2
        if cidx + 1 < _NCH:
            cps[nb] = pltpu.async_copy(
                feats_hbm.at[pl.ds((base + (cidx + 1) * _CH) * _D,
                                   _CH * _D)],
                bufs[nb], sems[nb])
        cps[cidx % 2].wait()
        buf = bufs[cidx % 2]
        coff = cidx * _CH

        def _row(r, carry, buf=buf, coff=coff):
            cur = carry[0]
            cnt = carry[1]
            sums = carry[2:2 + _DC]
            maxs = carry[2 + _DC:]
            sv = seg_sm[coff + r]
            changed = sv != cur

            @pl.when(changed)
            def _flush():
                for d in range(_DC):
                    asum_v[pl.ds(cur * _D + d * _NL, _NL)] = sums[d]
                    amax_v[pl.ds(cur * _D + d * _NL, _NL)] = maxs[d]
                acnt_v[pl.ds(cur * _NL, _NL)] = jnp.broadcast_to(
                    cnt, (_NL,))

            new_sums = []
            new_maxs = []
            for d in range(_DC):
                v = buf[pl.ds(r * _D + d * _NL, _NL)]
                new_sums.append(jnp.where(changed, v, sums[d] + v))
                new_maxs.append(jnp.where(changed, v,
                                          jnp.maximum(maxs[d], v)))
            cnt = jnp.where(changed, jnp.asarray(1.0, jnp.float32),
                            cnt + 1.0)
            return (sv, cnt, *new_sums, *new_maxs)

        carry = lax.fori_loop(0, _CH, _row, carry)

    cur = carry[0]
    cnt = carry[1]
    sums = carry[2:2 + _DC]
    maxs = carry[2 + _DC:]
    for d in range(_DC):
        asum_v[pl.ds(cur * _D + d * _NL, _NL)] = sums[d]
        amax_v[pl.ds(cur * _D + d * _NL, _NL)] = maxs[d]
    acnt_v[pl.ds(cur * _NL, _NL)] = jnp.broadcast_to(cnt, (_NL,))

    pltpu.sync_copy(asum_v, sums_hbm.at[wid])
    pltpu.sync_copy(amax_v, maxs_hbm.at[wid])
    pltpu.sync_copy(acnt_v, cnts_hbm.at[wid])


def _sc_reduce(feats, seg):
    feats_flat = feats.reshape(_N * _D)
    mesh = plsc.VectorSubcoreMesh(core_axis_name="c", subcore_axis_name="s")
    sums_p, maxs_p, cnts_p = pl.kernel(
        _sc_body,
        out_type=(
            jax.ShapeDtypeStruct((_NW, _B * _D), jnp.float32),
            jax.ShapeDtypeStruct((_NW, _B * _D), jnp.float32),
            jax.ShapeDtypeStruct((_NW, _B * _NL), jnp.float32),
        ),
        mesh=mesh,
        scratch_types=[
            pltpu.VMEM((_RPW,), jnp.int32),
            pltpu.SMEM((_RPW,), jnp.int32),
            pltpu.VMEM((_CH * _D,), jnp.float32),
            pltpu.VMEM((_CH * _D,), jnp.float32),
            pltpu.VMEM((_B * _D,), jnp.float32),
            pltpu.VMEM((_B * _D,), jnp.float32),
            pltpu.VMEM((_B * _NL,), jnp.float32),
            pltpu.SemaphoreType.DMA,
            pltpu.SemaphoreType.DMA,
        ],
    )(feats_flat, seg)
    return (sums_p.reshape(_NW, _B, _D), maxs_p.reshape(_NW, _B, _D),
            cnts_p.reshape(_NW, _B, _NL))


_BR = 2048
_T = _N // _BR


def _tc_body(feats_b, seg_b, sums_b, maxs_b, cnts_b,
             W1_b, b1_b, W2_b, b2_b, out_b, gate_s):
    i = pl.program_id(0)

    @pl.when(i == 0)
    def _gate():
        ssum = sums_b[0]
        smax = maxs_b[0]
        cnt = cnts_b[0]
        for w in range(1, _NW):
            ssum = ssum + sums_b[w]
            smax = jnp.maximum(smax, maxs_b[w])
            cnt = cnt + cnts_b[w]
        counts = cnt[:, 0:1]
        means = ssum / counts

        def mlp(v):
            h = jnp.maximum(
                jnp.dot(v, W1_b[...], preferred_element_type=jnp.float32)
                + b1_b[0, :][None, :], 0.0)
            return (jnp.dot(h, W2_b[...],
                            preferred_element_type=jnp.float32)
                    + b2_b[0, :][None, :])

        gate_s[...] = jax.nn.sigmoid(mlp(means) + mlp(smax))

    @pl.when(i > 0)
    def _apply():
        x = feats_b[...]
        segv = seg_b[...]
        onehot = (segv ==
                  lax.broadcasted_iota(jnp.int32, (_BR, _B), 1)
                  ).astype(jnp.float32)
        gtok = jnp.dot(onehot, gate_s[...],
                       preferred_element_type=jnp.float32)
        out_b[...] = x * gtok


def _tc_apply(feats, seg2, sums_p, maxs_p, cnts_p, W1, b1r, W2, b2r):
    return pl.pallas_call(
        _tc_body,
        grid=(_T + 1,),
        in_specs=[
            pl.BlockSpec((_BR, _D), lambda i: (jnp.maximum(i - 1, 0), 0)),
            pl.BlockSpec((_BR, 1), lambda i: (jnp.maximum(i - 1, 0), 0)),
            pl.BlockSpec((_NW, _B, _D), lambda i: (0, 0, 0)),
            pl.BlockSpec((_NW, _B, _D), lambda i: (0, 0, 0)),
            pl.BlockSpec((_NW, _B, _NL), lambda i: (0, 0, 0)),
            pl.BlockSpec((_D, _HID), lambda i: (0, 0)),
            pl.BlockSpec((1, _HID), lambda i: (0, 0)),
            pl.BlockSpec((_HID, _D), lambda i: (0, 0)),
            pl.BlockSpec((1, _D), lambda i: (0, 0)),
        ],
        out_specs=pl.BlockSpec(
            (_BR, _D), lambda i: (jnp.maximum(i - 1, 0), 0)),
        out_shape=jax.ShapeDtypeStruct((_N, _D), jnp.float32),
        scratch_shapes=[
            pltpu.VMEM((_B, _D), jnp.float32),
        ],
        compiler_params=pltpu.CompilerParams(
            dimension_semantics=("arbitrary",)),
    )(feats, seg2, sums_p, maxs_p, cnts_p, W1, b1r, W2, b2r)


def kernel(feats, segment_ids, W1, b1, W2, b2):
    seg = segment_ids.astype(jnp.int32)
    sums_p, maxs_p, cnts_p = _sc_reduce(feats, seg)
    return _tc_apply(feats, seg.reshape(_N, 1), sums_p, maxs_p, cnts_p,
                     W1, b1.reshape(1, _HID), W2, b2.reshape(1, _D))


# speedup vs baseline: 1.3807x; 1.3807x over previous
import jax
import jax.numpy as jnp
from jax import lax
from jax.experimental import pallas as pl
from jax.experimental.pallas import tpu as pltpu
from jax.experimental.pallas import tpu_sc as plsc

_N = 16384
_D = 256
_B = 8
_HID = _D // 4

_NW = 32
_RPW = _N // _NW
_CH = 128
_NCH = _RPW // _CH
_NL = 16
_DC = _D // _NL


def _sc_body(feats_hbm, seg_hbm, sums_hbm, maxs_hbm, cnts_hbm,
             seg_v, seg_sm, cnt_sm, buf0, buf1, asum_v, amax_v, acnt_v,
             sem0, sem1):
    cc = lax.axis_index("c")
    ss = lax.axis_index("s")
    wid = ss * 2 + cc
    base = wid * _RPW

    pltpu.sync_copy(seg_hbm.at[pl.ds(base, _RPW)], seg_v)

    bufs = (buf0, buf1)
    sems = (sem0, sem1)
    cps = [None, None]
    cps[0] = pltpu.async_copy(
        feats_hbm.at[pl.ds(base, _CH), :], buf0, sem0)

    def _seg_to_smem(g, carry):
        vv = seg_v[pl.ds(g * _NL, _NL)]
        for k in range(_NL):
            seg_sm[g * _NL + k] = vv[k]
        return carry
    lax.fori_loop(0, _RPW // _NL, _seg_to_smem, 0)
    for b in range(_B):
        cnt_sm[b] = 0
    def _cnt(r, carry):
        s = seg_sm[r]
        cnt_sm[s] = cnt_sm[s] + 1
        return carry
    lax.fori_loop(0, _RPW, _cnt, 0)
    cnt_sc = [cnt_sm[b] for b in range(_B)]
    starts = []
    acc = 0
    for b in range(_B):
        starts.append(acc)
        acc = acc + cnt_sc[b]

    def _init(i, carry):
        asum_v[pl.ds(i * _NL, _NL)] = jnp.zeros((_NL,), jnp.float32)
        amax_v[pl.ds(i * _NL, _NL)] = jnp.full((_NL,), -jnp.inf,
                                               jnp.float32)
        return carry
    lax.fori_loop(0, _B * _DC, _init, 0)
    for b in range(_B):
        acnt_v[pl.ds(b * _NL, _NL)] = jnp.broadcast_to(
            jnp.asarray(cnt_sc[b], jnp.float32), (_NL,))

    zero16 = jnp.zeros((_NL,), jnp.float32)
    ninf16 = jnp.full((_NL,), -jnp.inf, jnp.float32)

    for cidx in range(_NCH):
        nb = (cidx + 1) % 2
        if cidx + 1 < _NCH:
            cps[nb] = pltpu.async_copy(
                feats_hbm.at[pl.ds(base + (cidx + 1) * _CH, _CH), :],
                bufs[nb], sems[nb])
        cps[cidx % 2].wait()
        buf = bufs[cidx % 2]
        coff = cidx * _CH

        for b in range(_B):
            lo = jnp.maximum(starts[b], coff) - coff
            hi = jnp.minimum(starts[b] + cnt_sc[b], coff + _CH) - coff
            n = hi - lo

            def _run(i, carry, buf=buf, lo=lo):
                sums = carry[:_DC]
                maxs = carry[_DC:]
                out = []
                outm = []
                for d in range(_DC):
                    v = buf[lo + i, pl.ds(d * _NL, _NL)]
                    out.append(sums[d] + v)
                    outm.append(jnp.maximum(maxs[d], v))
                return (*out, *outm)

            res = lax.fori_loop(0, jnp.maximum(n, 0), _run,
                                (*([zero16] * _DC), *([ninf16] * _DC)))

            @pl.when(n > 0)
            def _flush(res=res, b=b):
                for d in range(_DC):
                    off = b * _D + d * _NL
                    asum_v[pl.ds(off, _NL)] = (
                        asum_v[pl.ds(off, _NL)] + res[d])
                    amax_v[pl.ds(off, _NL)] = jnp.maximum(
                        amax_v[pl.ds(off, _NL)], res[_DC + d])

    pltpu.sync_copy(asum_v, sums_hbm.at[wid])
    pltpu.sync_copy(amax_v, maxs_hbm.at[wid])
    pltpu.sync_copy(acnt_v, cnts_hbm.at[wid])


def _sc_reduce(feats, seg):
    mesh = plsc.VectorSubcoreMesh(core_axis_name="c", subcore_axis_name="s")
    sums_p, maxs_p, cnts_p = pl.kernel(
        _sc_body,
        out_type=(
            jax.ShapeDtypeStruct((_NW, _B * _D), jnp.float32),
            jax.ShapeDtypeStruct((_NW, _B * _D), jnp.float32),
            jax.ShapeDtypeStruct((_NW, _B * _NL), jnp.float32),
        ),
        mesh=mesh,
        scratch_types=[
            pltpu.VMEM((_RPW,), jnp.int32),
            pltpu.SMEM((_RPW,), jnp.int32),
            pltpu.SMEM((_B,), jnp.int32),
            pltpu.VMEM((_CH, _D), jnp.float32),
            pltpu.VMEM((_CH, _D), jnp.float32),
            pltpu.VMEM((_B * _D,), jnp.float32),
            pltpu.VMEM((_B * _D,), jnp.float32),
            pltpu.VMEM((_B * _NL,), jnp.float32),
            pltpu.SemaphoreType.DMA,
            pltpu.SemaphoreType.DMA,
        ],
    )(feats, seg)
    return (sums_p.reshape(_NW, _B, _D), maxs_p.reshape(_NW, _B, _D),
            cnts_p.reshape(_NW, _B, _NL))


_BR = 2048
_T = _N // _BR


def _tc_body(feats_b, seg_b, sums_b, maxs_b, cnts_b,
             W1_b, b1_b, W2_b, b2_b, out_b, gate_s):
    i = pl.program_id(0)

    @pl.when(i == 0)
    def _gate():
        ssum = sums_b[0]
        smax = maxs_b[0]
        cnt = cnts_b[0]
        for w in range(1, _NW):
            ssum = ssum + sums_b[w]
            smax = jnp.maximum(smax, maxs_b[w])
            cnt = cnt + cnts_b[w]
        counts = cnt[:, 0:1]
        means = ssum / counts

        def mlp(v):
            h = jnp.maximum(
                jnp.dot(v, W1_b[...], preferred_element_type=jnp.float32)
                + b1_b[0, :][None, :], 0.0)
            return (jnp.dot(h, W2_b[...],
                            preferred_element_type=jnp.float32)
                    + b2_b[0, :][None, :])

        gate_s[...] = jax.nn.sigmoid(mlp(means) + mlp(smax))

    @pl.when(i > 0)
    def _apply():
        x = feats_b[...]
        segv = seg_b[...]
        onehot = (segv ==
                  lax.broadcasted_iota(jnp.int32, (_BR, _B), 1)
                  ).astype(jnp.float32)
        gtok = jnp.dot(onehot, gate_s[...],
                       preferred_element_type=jnp.float32)
        out_b[...] = x * gtok


def _tc_apply(feats, seg2, sums_p, maxs_p, cnts_p, W1, b1r, W2, b2r):
    return pl.pallas_call(
        _tc_body,
        grid=(_T + 1,),
        in_specs=[
            pl.BlockSpec((_BR, _D), lambda i: (jnp.maximum(i - 1, 0), 0)),
            pl.BlockSpec((_BR, 1), lambda i: (jnp.maximum(i - 1, 0), 0)),
            pl.BlockSpec((_NW, _B, _D), lambda i: (0, 0, 0)),
            pl.BlockSpec((_NW, _B, _D), lambda i: (0, 0, 0)),
            pl.BlockSpec((_NW, _B, _NL), lambda i: (0, 0, 0)),
            pl.BlockSpec((_D, _HID), lambda i: (0, 0)),
            pl.BlockSpec((1, _HID), lambda i: (0, 0)),
            pl.BlockSpec((_HID, _D), lambda i: (0, 0)),
            pl.BlockSpec((1, _D), lambda i: (0, 0)),
        ],
        out_specs=pl.BlockSpec(
            (_BR, _D), lambda i: (jnp.maximum(i - 1, 0), 0)),
        out_shape=jax.ShapeDtypeStruct((_N, _D), jnp.float32),
        scratch_shapes=[
            pltpu.VMEM((_B, _D), jnp.float32),
        ],
        compiler_params=pltpu.CompilerParams(
            dimension_semantics=("arbitrary",)),
    )(feats, seg2, sums_p, maxs_p, cnts_p, W1, b1r, W2, b2r)


def kernel(feats, segment_ids, W1, b1, W2, b2):
    seg = segment_ids.astype(jnp.int32)
    sums_p, maxs_p, cnts_p = _sc_reduce(feats, seg)
    return _tc_apply(feats, seg.reshape(_N, 1), sums_p, maxs_p, cnts_p,
                     W1, b1.reshape(1, _HID), W2, b2.reshape(1, _D))
